# baseline (device time: 57731 ns/iter reference)
import functools

import jax
import jax.numpy as jnp
from jax import lax
from jax.experimental import pallas as pl
from jax.experimental.pallas import tpu as pltpu

N_DEV = 8
BLOCK_M = 1024
SUB_M = 512
DISTS = (1, 2, 4)


def kernel(x):
    m_per, n = x.shape
    n_blocks = m_per // BLOCK_M

    def body(
        x_ref,
        out_ref,
        prefix_out_ref,
        carry_ref,
        prefix_ref,
        acc_ref,
        comm_ref,
        tril_ref,
        send_sems,
        recv_sems,
    ):
        b = pl.program_id(0)
        my = lax.axis_index("i")

        @pl.when(b == 0)
        def _():
            carry_ref[...] = jnp.ones((1, n), jnp.float32)
            row = lax.broadcasted_iota(jnp.int32, (SUB_M, SUB_M), 0)
            col = lax.broadcasted_iota(jnp.int32, (SUB_M, SUB_M), 1)
            tril_ref[...] = (row >= col).astype(jnp.bfloat16)

        for h in range(BLOCK_M // SUB_M):
            y = x_ref[pl.ds(h * SUB_M, SUB_M), :]
            lx = jnp.log(y).astype(jnp.bfloat16)
            cs = jnp.dot(tril_ref[...], lx, preferred_element_type=jnp.float32)
            ex = jnp.exp(cs) * carry_ref[...]
            out_ref[pl.ds(h * SUB_M, SUB_M), :] = ex
            carry_ref[...] = ex[SUB_M - 1 : SUB_M, :]

        def partner_barrier(sem):
            for d in DISTS:

                @pl.when(my + d < N_DEV)
                def _():
                    pl.semaphore_signal(
                        sem,
                        inc=1,
                        device_id=(my + d,),
                        device_id_type=pl.DeviceIdType.MESH,
                    )

                @pl.when(my - d >= 0)
                def _():
                    pl.semaphore_signal(
                        sem,
                        inc=1,
                        device_id=(my - d,),
                        device_id_type=pl.DeviceIdType.MESH,
                    )

            for d in DISTS:

                @pl.when(my + d < N_DEV)
                def _():
                    pl.semaphore_wait(sem, 1)

                @pl.when(my - d >= 0)
                def _():
                    pl.semaphore_wait(sem, 1)

        @pl.when(b == n_blocks - 1)
        def _():
            partner_barrier(pltpu.get_barrier_semaphore())

            prefix_ref[...] = jnp.ones((1, n), jnp.float32)
            acc_ref[...] = carry_ref[...]
            for r, d in enumerate(DISTS):

                @pl.when(my + d < N_DEV)
                def _():
                    send = pltpu.make_async_remote_copy(
                        src_ref=acc_ref,
                        dst_ref=comm_ref.at[r],
                        send_sem=send_sems.at[r],
                        recv_sem=recv_sems.at[r],
                        device_id=(my + d,),
                        device_id_type=pl.DeviceIdType.MESH,
                    )
                    send.start()
                    send.wait_send()

                @pl.when(my >= d)
                def _():
                    recv = pltpu.make_async_remote_copy(
                        src_ref=acc_ref,
                        dst_ref=comm_ref.at[r],
                        send_sem=send_sems.at[r],
                        recv_sem=recv_sems.at[r],
                        device_id=(my - d,),
                        device_id_type=pl.DeviceIdType.MESH,
                    )
                    recv.wait_recv()
                    prefix_ref[...] = prefix_ref[...] * comm_ref[r]
                    acc_ref[...] = acc_ref[...] * comm_ref[r]

            prefix_out_ref[...] = prefix_ref[...]

            @functools.partial(pl.run_scoped, exit_sem=pltpu.SemaphoreType.REGULAR)
            def _(exit_sem):
                partner_barrier(exit_sem)

    unscaled, prefix = pl.pallas_call(
        body,
        grid=(n_blocks,),
        in_specs=[pl.BlockSpec((BLOCK_M, n), lambda b: (b, 0))],
        out_specs=[
            pl.BlockSpec((BLOCK_M, n), lambda b: (b, 0)),
            pl.BlockSpec((1, n), lambda b: (0, 0)),
        ],
        out_shape=[
            jax.ShapeDtypeStruct((m_per, n), jnp.float32),
            jax.ShapeDtypeStruct((1, n), jnp.float32),
        ],
        scratch_shapes=[
            pltpu.VMEM((1, n), jnp.float32),
            pltpu.VMEM((1, n), jnp.float32),
            pltpu.VMEM((1, n), jnp.float32),
            pltpu.VMEM((3, 1, n), jnp.float32),
            pltpu.VMEM((SUB_M, SUB_M), jnp.bfloat16),
            pltpu.SemaphoreType.DMA((3,)),
            pltpu.SemaphoreType.DMA((3,)),
        ],
        compiler_params=pltpu.CompilerParams(
            dimension_semantics=("arbitrary",),
            vmem_limit_bytes=60 * 1024 * 1024,
            collective_id=0,
        ),
    )(x)
    return unscaled * prefix


# device time: 57057 ns/iter; 1.0118x vs baseline; 1.0118x over previous
import functools

import jax
import jax.numpy as jnp
from jax import lax
from jax.experimental import pallas as pl
from jax.experimental.pallas import tpu as pltpu

N_DEV = 8
BLOCK_M = 1024
SUB_M = 512
DISTS = (1, 2, 4)


def kernel(x):
    m_per, n = x.shape
    n_blocks = m_per // BLOCK_M

    def body(
        x_ref,
        out_ref,
        prefix_out_ref,
        carry_ref,
        prefix_ref,
        acc_ref,
        comm_ref,
        tril_ref,
        send_sems,
        recv_sems,
    ):
        b = pl.program_id(0)
        my = lax.axis_index("i")

        @pl.when(b == 0)
        def _():
            carry_ref[...] = jnp.ones((1, n), jnp.float32)
            row = lax.broadcasted_iota(jnp.int32, (SUB_M, SUB_M), 0)
            col = lax.broadcasted_iota(jnp.int32, (SUB_M, SUB_M), 1)
            tril_ref[...] = (row >= col).astype(jnp.bfloat16)

        for h in range(BLOCK_M // SUB_M):
            y = x_ref[pl.ds(h * SUB_M, SUB_M), :]
            lx = jnp.log(y).astype(jnp.bfloat16)
            cs = jnp.dot(tril_ref[...], lx, preferred_element_type=jnp.float32)
            out_ref[pl.ds(h * SUB_M, SUB_M), :] = jnp.exp(cs) * carry_ref[...]
            p = y
            m = SUB_M
            while m > 1:
                m //= 2
                p = p[:m, :] * p[m:, :]
            carry_ref[...] = carry_ref[...] * p

        def partner_barrier(sem):
            for d in DISTS:

                @pl.when(my + d < N_DEV)
                def _():
                    pl.semaphore_signal(
                        sem,
                        inc=1,
                        device_id=(my + d,),
                        device_id_type=pl.DeviceIdType.MESH,
                    )

                @pl.when(my - d >= 0)
                def _():
                    pl.semaphore_signal(
                        sem,
                        inc=1,
                        device_id=(my - d,),
                        device_id_type=pl.DeviceIdType.MESH,
                    )

            for d in DISTS:

                @pl.when(my + d < N_DEV)
                def _():
                    pl.semaphore_wait(sem, 1)

                @pl.when(my - d >= 0)
                def _():
                    pl.semaphore_wait(sem, 1)

        @pl.when(b == n_blocks - 1)
        def _():
            partner_barrier(pltpu.get_barrier_semaphore())

            prefix_ref[...] = jnp.ones((1, n), jnp.float32)
            acc_ref[...] = carry_ref[...]
            for r, d in enumerate(DISTS):

                @pl.when(my + d < N_DEV)
                def _():
                    send = pltpu.make_async_remote_copy(
                        src_ref=acc_ref,
                        dst_ref=comm_ref.at[r],
                        send_sem=send_sems.at[r],
                        recv_sem=recv_sems.at[r],
                        device_id=(my + d,),
                        device_id_type=pl.DeviceIdType.MESH,
                    )
                    send.start()
                    send.wait_send()

                @pl.when(my >= d)
                def _():
                    recv = pltpu.make_async_remote_copy(
                        src_ref=acc_ref,
                        dst_ref=comm_ref.at[r],
                        send_sem=send_sems.at[r],
                        recv_sem=recv_sems.at[r],
                        device_id=(my - d,),
                        device_id_type=pl.DeviceIdType.MESH,
                    )
                    recv.wait_recv()
                    prefix_ref[...] = prefix_ref[...] * comm_ref[r]
                    acc_ref[...] = acc_ref[...] * comm_ref[r]

            prefix_out_ref[...] = prefix_ref[...]

            @functools.partial(pl.run_scoped, exit_sem=pltpu.SemaphoreType.REGULAR)
            def _(exit_sem):
                partner_barrier(exit_sem)

    unscaled, prefix = pl.pallas_call(
        body,
        grid=(n_blocks,),
        in_specs=[pl.BlockSpec((BLOCK_M, n), lambda b: (b, 0))],
        out_specs=[
            pl.BlockSpec((BLOCK_M, n), lambda b: (b, 0)),
            pl.BlockSpec((1, n), lambda b: (0, 0)),
        ],
        out_shape=[
            jax.ShapeDtypeStruct((m_per, n), jnp.float32),
            jax.ShapeDtypeStruct((1, n), jnp.float32),
        ],
        scratch_shapes=[
            pltpu.VMEM((1, n), jnp.float32),
            pltpu.VMEM((1, n), jnp.float32),
            pltpu.VMEM((1, n), jnp.float32),
            pltpu.VMEM((3, 1, n), jnp.float32),
            pltpu.VMEM((SUB_M, SUB_M), jnp.bfloat16),
            pltpu.SemaphoreType.DMA((3,)),
            pltpu.SemaphoreType.DMA((3,)),
        ],
        compiler_params=pltpu.CompilerParams(
            dimension_semantics=("arbitrary",),
            vmem_limit_bytes=60 * 1024 * 1024,
            collective_id=0,
        ),
    )(x)
    return unscaled * prefix
